# baseline (speedup 1.0000x reference)
import functools

import jax
import jax.numpy as jnp
from jax import lax
from jax.experimental import pallas as pl
from jax.experimental.pallas import tpu as pltpu
from jax.experimental.pallas import tpu_sc as plsc

_BETA = 1.0 / 9
_NEG_POS_RATIO = 3
_LANES = 128
_BR = 512
_NW = 32
_BINS = 65536


def _imin():
    return jnp.int32(-2147483648)


def _imaxp():
    return jnp.int32(0x7FFFFFFF)


def _softplus(x):
    return jnp.maximum(x, 0.0) + jnp.log1p(jnp.exp(-jnp.abs(x)))


def _a_body(n_total, grid, c0, c1, lab, p1, p3, g1, g3, v_out, acc, k_out):
    step = pl.program_id(0)
    d = c1[...] - c0[...]
    bits = jax.lax.bitcast_convert_type(d, jnp.int32)
    v = jnp.where(bits >= 0, bits, bits ^ _imaxp())
    pos = lab[...] > 0
    v_out[...] = jnp.where(pos, _imin(), v)
    posf = pos.astype(jnp.float32)
    ce_pos = _softplus(-d)
    a1 = jnp.abs(p1[...] - g1[...])
    a3 = jnp.abs(p3[...] - g3[...])
    sl1 = jnp.where(a1 < _BETA, 0.5 * a1 * a1 / _BETA, a1 - 0.5 * _BETA)
    sl3 = jnp.where(a3 < _BETA, 0.5 * a3 * a3 / _BETA, a3 - 0.5 * _BETA)
    pcnt = jnp.sum(posf, axis=0)
    spos = jnp.sum(ce_pos * posf, axis=0)
    verp = jnp.sum((sl1 + sl3) * posf, axis=0)
    rows = jax.lax.broadcasted_iota(jnp.int32, (8, _LANES), 0)
    part = (
        jnp.where(rows == 0, pcnt[None, :], 0.0)
        + jnp.where(rows == 1, spos[None, :], 0.0)
        + jnp.where(rows == 2, verp[None, :], 0.0)
    )

    @pl.when(step == 0)
    def _():
        acc[...] = part

    @pl.when(step != 0)
    def _():
        acc[...] = acc[...] + part

    @pl.when(step == grid - 1)
    def _():
        npos = jnp.sum(acc[0, :]).astype(jnp.int32)
        k_out[0] = jnp.minimum(npos * _NEG_POS_RATIO, jnp.int32(n_total) - npos)


def _h1_body(n_total, v_hbm, zeros_hbm, out_hbm, data_v, hist_v):
    ch = n_total // _NW
    wid = lax.axis_index("s") * 2 + lax.axis_index("c")
    pltpu.sync_copy(v_hbm.at[pl.ds(wid * ch, ch)], data_v)
    pltpu.sync_copy(zeros_hbm, hist_v)
    ones16 = jnp.full((16,), 1, jnp.int32)
    m31 = jnp.full((16,), -2147483648, jnp.int32)
    sh16 = jnp.full((16,), 16, jnp.int32)

    def hbody(i, carry):
        x = data_v[pl.ds(i * 16, 16)]
        u = lax.bitwise_xor(x, m31)
        hi = lax.shift_right_logical(u, sh16)
        plsc.addupdate_scatter(hist_v, [hi], ones16)
        return carry

    lax.fori_loop(0, ch // 16, hbody, 0)
    pltpu.sync_copy(hist_v, out_hbm.at[wid])


def _h2_body(n_total, v_hbm, zeros_hbm, b_hbm, out_hbm, data_v, hist_v, b_v):
    ch = n_total // _NW
    wid = lax.axis_index("s") * 2 + lax.axis_index("c")
    pltpu.sync_copy(v_hbm.at[pl.ds(wid * ch, ch)], data_v)
    pltpu.sync_copy(zeros_hbm, hist_v)
    pltpu.sync_copy(b_hbm.at[pl.ds(0, 16)], b_v)
    bb = b_v[...]
    ones16 = jnp.full((16,), 1, jnp.int32)
    m31 = jnp.full((16,), -2147483648, jnp.int32)
    sh16 = jnp.full((16,), 16, jnp.int32)
    mlow = jnp.full((16,), 0xFFFF, jnp.int32)

    def hbody(i, carry):
        x = data_v[pl.ds(i * 16, 16)]
        u = lax.bitwise_xor(x, m31)
        hi = lax.shift_right_logical(u, sh16)
        low = lax.bitwise_and(u, mlow)
        plsc.addupdate_scatter(hist_v, [low], ones16, mask=hi == bb)
        return carry

    lax.fori_loop(0, ch // 16, hbody, 0)
    pltpu.sync_copy(hist_v, out_hbm.at[wid])


def _sc_hist(body, n_total, *args):
    mesh = plsc.VectorSubcoreMesh(core_axis_name="c", subcore_axis_name="s")
    ch = n_total // _NW
    scratch = [
        pltpu.VMEM((ch,), jnp.int32),
        pltpu.VMEM((_BINS,), jnp.int32),
    ]
    if body is _h2_body:
        scratch.append(pltpu.VMEM((16,), jnp.int32))
    return pl.kernel(
        functools.partial(body, n_total),
        mesh=mesh,
        out_type=jax.ShapeDtypeStruct((_NW, _BINS), jnp.int32),
        scratch_types=scratch,
        compiler_params=pltpu.CompilerParams(needs_layout_passes=False),
    )(*args)


def _suffix_g(h_i32):
    m2 = jnp.sum(h_i32.astype(jnp.float32), axis=0)
    ii = jax.lax.broadcasted_iota(jnp.int32, (_LANES, _LANES), 0)
    jj = jax.lax.broadcasted_iota(jnp.int32, (_LANES, _LANES), 1)
    tincl = (ii >= jj).astype(jnp.float32)
    gw = jnp.dot(m2, tincl, preferred_element_type=jnp.float32)
    i5 = jax.lax.broadcasted_iota(jnp.int32, (512, 512), 0)
    j5 = jax.lax.broadcasted_iota(jnp.int32, (512, 512), 1)
    ltr = (j5 > i5).astype(jnp.float32)
    rowtot = gw[:, 0:1]
    sgt = jnp.dot(ltr, rowtot, preferred_element_type=jnp.float32)
    return m2, sgt + gw


def _bin_idx():
    ii = jax.lax.broadcasted_iota(jnp.int32, (512, _LANES), 0)
    jj = jax.lax.broadcasted_iota(jnp.int32, (512, _LANES), 1)
    return ii * _LANES + jj


def _f1_body(h_ref, k_ref, bvec_out, bstar_out, chi_out):
    kf = k_ref[0].astype(jnp.float32)
    m2, g = _suffix_g(h_ref[...])
    idx = _bin_idx()
    bstar = jnp.max(jnp.where(g >= kf, idx, -1))
    chi = jnp.sum(jnp.where(idx > bstar, m2, 0.0)).astype(jnp.int32)
    bvec_out[...] = jnp.full((8, _LANES), bstar, jnp.int32)
    bstar_out[0] = bstar
    chi_out[0] = chi


def _bf_body(v_ref, acc_ref, h2_ref, k_ref, bstar_ref, chi_ref, o_total, o_cls, o_ver):
    acc = acc_ref[...]
    npos_f = jnp.sum(acc[0, :])
    s_pos = jnp.sum(acc[1, :])
    ver_sum = jnp.sum(acc[2, :])
    k = k_ref[0]
    k2f = (k - chi_ref[0]).astype(jnp.float32)
    _, g2 = _suffix_g(h2_ref[...])
    idx = _bin_idx()
    low = jnp.max(jnp.where(g2 >= k2f, idx, -1))
    t_u = (bstar_ref[0] << 16) | low
    t_i = t_u ^ _imin()
    varr = v_ref[...]
    sel = varr > t_i
    eq = varr == t_i
    cnt_gt = jnp.sum(sel.astype(jnp.int32))
    u = varr ^ _imin()
    bits_f = jnp.where(u < 0, u & _imaxp(), jnp.bitwise_not(u))
    dd = jax.lax.bitcast_convert_type(bits_f, jnp.float32)
    ml = _softplus(dd)
    s_main = jnp.sum(jnp.where(sel, ml, 0.0))
    s_eq = jnp.sum(jnp.where(eq, ml, 0.0))
    c_eq = jnp.sum(eq.astype(jnp.float32))
    mlt = s_eq / c_eq
    s_sel = s_main + (k - cnt_gt).astype(jnp.float32) * mlt
    s_sel = jnp.where(k > 0, s_sel, 0.0)
    denom = jnp.maximum((npos_f + k.astype(jnp.float32)), 1.0)
    cls = jnp.clip((s_pos + s_sel) / denom, 0.0, 5.0)
    ver = jnp.clip(ver_sum / jnp.maximum(2.0 * npos_f, 1.0), 0.0, 5.0)
    o_cls[0] = cls
    o_ver[0] = ver
    o_total[0] = cls + ver


def kernel(confidence, predicted_locations, labels, gt_locations):
    b, a = labels.shape
    n = b * a
    nr = n // _LANES
    grid = nr // _BR
    conf = confidence.reshape(n, 2)
    c0 = conf[:, 0].reshape(nr, _LANES)
    c1 = conf[:, 1].reshape(nr, _LANES)
    pl4 = predicted_locations.reshape(n, 4)
    gl4 = gt_locations.reshape(n, 4)
    p1 = pl4[:, 1].reshape(nr, _LANES)
    p3 = pl4[:, 3].reshape(nr, _LANES)
    g1 = gl4[:, 1].reshape(nr, _LANES)
    g3 = gl4[:, 3].reshape(nr, _LANES)
    lab = labels.reshape(nr, _LANES)

    row_spec = pl.BlockSpec((_BR, _LANES), lambda i: (i, 0))
    acc_spec = pl.BlockSpec((8, _LANES), lambda i: (0, 0))
    smem_spec = pl.BlockSpec(memory_space=pltpu.SMEM)
    v, acc, kk = pl.pallas_call(
        functools.partial(_a_body, n, grid),
        grid=(grid,),
        in_specs=[row_spec] * 7,
        out_specs=[row_spec, acc_spec, smem_spec],
        out_shape=[
            jax.ShapeDtypeStruct((nr, _LANES), jnp.int32),
            jax.ShapeDtypeStruct((8, _LANES), jnp.float32),
            jax.ShapeDtypeStruct((1,), jnp.int32),
        ],
    )(c0, c1, lab, p1, p3, g1, g3)

    vflat = v.reshape(n)
    zeros_bins = jnp.zeros((_BINS,), jnp.int32)
    h1 = _sc_hist(_h1_body, n, vflat, zeros_bins)

    vmem_spec = pl.BlockSpec(memory_space=pltpu.VMEM)
    bvec, bstar, chi = pl.pallas_call(
        _f1_body,
        in_specs=[vmem_spec, smem_spec],
        out_specs=[vmem_spec, smem_spec, smem_spec],
        out_shape=[
            jax.ShapeDtypeStruct((8, _LANES), jnp.int32),
            jax.ShapeDtypeStruct((1,), jnp.int32),
            jax.ShapeDtypeStruct((1,), jnp.int32),
        ],
    )(h1.reshape(_NW, 512, _LANES), kk)

    h2 = _sc_hist(_h2_body, n, vflat, zeros_bins, bvec.reshape(8 * _LANES))

    total, cls, ver = pl.pallas_call(
        _bf_body,
        in_specs=[vmem_spec, vmem_spec, vmem_spec, smem_spec, smem_spec, smem_spec],
        out_specs=[smem_spec, smem_spec, smem_spec],
        out_shape=[
            jax.ShapeDtypeStruct((1,), jnp.float32),
            jax.ShapeDtypeStruct((1,), jnp.float32),
            jax.ShapeDtypeStruct((1,), jnp.float32),
        ],
    )(v, acc, h2.reshape(_NW, 512, _LANES), kk, bstar, chi)

    z = jnp.zeros((), jnp.float32)
    return (total.reshape(()), cls.reshape(()), ver.reshape(()), z)


# speedup vs baseline: 1.0021x; 1.0021x over previous
import functools

import jax
import jax.numpy as jnp
from jax import lax
from jax.experimental import pallas as pl
from jax.experimental.pallas import tpu as pltpu
from jax.experimental.pallas import tpu_sc as plsc

_BETA = 1.0 / 9
_NEG_POS_RATIO = 3
_LANES = 128
_BR = 512
_NW = 32
_BINS = 65536


def _imin():
    return jnp.int32(-2147483648)


def _imaxp():
    return jnp.int32(0x7FFFFFFF)


def _softplus(x):
    return jnp.maximum(x, 0.0) + jnp.log1p(jnp.exp(-jnp.abs(x)))


def _a_body(n_total, grid, c0, c1, lab, p1, p3, g1, g3, v_out, acc, k_out):
    step = pl.program_id(0)
    d = c1[...] - c0[...]
    bits = jax.lax.bitcast_convert_type(d, jnp.int32)
    v = jnp.where(bits >= 0, bits, bits ^ _imaxp())
    pos = lab[...] > 0
    v_out[...] = jnp.where(pos, _imin(), v)
    posf = pos.astype(jnp.float32)
    ce_pos = _softplus(-d)
    a1 = jnp.abs(p1[...] - g1[...])
    a3 = jnp.abs(p3[...] - g3[...])
    sl1 = jnp.where(a1 < _BETA, 0.5 * a1 * a1 / _BETA, a1 - 0.5 * _BETA)
    sl3 = jnp.where(a3 < _BETA, 0.5 * a3 * a3 / _BETA, a3 - 0.5 * _BETA)
    pcnt = jnp.sum(posf, axis=0)
    spos = jnp.sum(ce_pos * posf, axis=0)
    verp = jnp.sum((sl1 + sl3) * posf, axis=0)
    rows = jax.lax.broadcasted_iota(jnp.int32, (8, _LANES), 0)
    part = (
        jnp.where(rows == 0, pcnt[None, :], 0.0)
        + jnp.where(rows == 1, spos[None, :], 0.0)
        + jnp.where(rows == 2, verp[None, :], 0.0)
    )

    @pl.when(step == 0)
    def _():
        acc[...] = part

    @pl.when(step != 0)
    def _():
        acc[...] = acc[...] + part

    @pl.when(step == grid - 1)
    def _():
        npos = jnp.sum(acc[0, :]).astype(jnp.int32)
        k_out[0] = jnp.minimum(npos * _NEG_POS_RATIO, jnp.int32(n_total) - npos)


_UNROLL = 8


def _h1_body(n_total, v_hbm, zeros_hbm, out_hbm, data_v, hist_v):
    ch = n_total // _NW
    wid = lax.axis_index("s") * 2 + lax.axis_index("c")
    pltpu.sync_copy(v_hbm.at[pl.ds(wid * ch, ch)], data_v)
    pltpu.sync_copy(zeros_hbm, hist_v)
    ones16 = jnp.full((16,), 1, jnp.int32)
    m31 = jnp.full((16,), -2147483648, jnp.int32)
    sh16 = jnp.full((16,), 16, jnp.int32)

    def hbody(i, carry):
        base = i * (16 * _UNROLL)
        for t in range(_UNROLL):
            x = data_v[pl.ds(base + t * 16, 16)]
            u = lax.bitwise_xor(x, m31)
            hi = lax.shift_right_logical(u, sh16)
            plsc.addupdate_scatter(hist_v, [hi], ones16)
        return carry

    lax.fori_loop(0, ch // (16 * _UNROLL), hbody, 0)
    pltpu.sync_copy(hist_v, out_hbm.at[wid])


def _h2_body(n_total, v_hbm, zeros_hbm, b_hbm, out_hbm, data_v, hist_v, b_v):
    ch = n_total // _NW
    wid = lax.axis_index("s") * 2 + lax.axis_index("c")
    pltpu.sync_copy(v_hbm.at[pl.ds(wid * ch, ch)], data_v)
    pltpu.sync_copy(zeros_hbm, hist_v)
    pltpu.sync_copy(b_hbm.at[pl.ds(0, 16)], b_v)
    bb = b_v[...]
    ones16 = jnp.full((16,), 1, jnp.int32)
    m31 = jnp.full((16,), -2147483648, jnp.int32)
    sh16 = jnp.full((16,), 16, jnp.int32)
    mlow = jnp.full((16,), 0xFFFF, jnp.int32)

    def hbody(i, carry):
        base = i * (16 * _UNROLL)
        for t in range(_UNROLL):
            x = data_v[pl.ds(base + t * 16, 16)]
            u = lax.bitwise_xor(x, m31)
            hi = lax.shift_right_logical(u, sh16)
            low = lax.bitwise_and(u, mlow)
            plsc.addupdate_scatter(hist_v, [low], ones16, mask=hi == bb)
        return carry

    lax.fori_loop(0, ch // (16 * _UNROLL), hbody, 0)
    pltpu.sync_copy(hist_v, out_hbm.at[wid])


def _sc_hist(body, n_total, *args):
    mesh = plsc.VectorSubcoreMesh(core_axis_name="c", subcore_axis_name="s")
    ch = n_total // _NW
    scratch = [
        pltpu.VMEM((ch,), jnp.int32),
        pltpu.VMEM((_BINS,), jnp.int32),
    ]
    if body is _h2_body:
        scratch.append(pltpu.VMEM((16,), jnp.int32))
    return pl.kernel(
        functools.partial(body, n_total),
        mesh=mesh,
        out_type=jax.ShapeDtypeStruct((_NW, _BINS), jnp.int32),
        scratch_types=scratch,
        compiler_params=pltpu.CompilerParams(needs_layout_passes=False),
    )(*args)


def _suffix_g(h_i32):
    m2 = jnp.sum(h_i32.astype(jnp.float32), axis=0)
    ii = jax.lax.broadcasted_iota(jnp.int32, (_LANES, _LANES), 0)
    jj = jax.lax.broadcasted_iota(jnp.int32, (_LANES, _LANES), 1)
    tincl = (ii >= jj).astype(jnp.float32)
    gw = jnp.dot(m2, tincl, preferred_element_type=jnp.float32)
    i5 = jax.lax.broadcasted_iota(jnp.int32, (512, 512), 0)
    j5 = jax.lax.broadcasted_iota(jnp.int32, (512, 512), 1)
    ltr = (j5 > i5).astype(jnp.float32)
    rowtot = gw[:, 0:1]
    sgt = jnp.dot(ltr, rowtot, preferred_element_type=jnp.float32)
    return m2, sgt + gw


def _bin_idx():
    ii = jax.lax.broadcasted_iota(jnp.int32, (512, _LANES), 0)
    jj = jax.lax.broadcasted_iota(jnp.int32, (512, _LANES), 1)
    return ii * _LANES + jj


def _f1_body(h_ref, k_ref, bvec_out, bstar_out, chi_out):
    kf = k_ref[0].astype(jnp.float32)
    m2, g = _suffix_g(h_ref[...])
    idx = _bin_idx()
    bstar = jnp.max(jnp.where(g >= kf, idx, -1))
    chi = jnp.sum(jnp.where(idx > bstar, m2, 0.0)).astype(jnp.int32)
    bvec_out[...] = jnp.full((8, _LANES), bstar, jnp.int32)
    bstar_out[0] = bstar
    chi_out[0] = chi


def _bf_body(v_ref, acc_ref, h2_ref, k_ref, bstar_ref, chi_ref, o_total, o_cls, o_ver):
    acc = acc_ref[...]
    npos_f = jnp.sum(acc[0, :])
    s_pos = jnp.sum(acc[1, :])
    ver_sum = jnp.sum(acc[2, :])
    k = k_ref[0]
    k2f = (k - chi_ref[0]).astype(jnp.float32)
    _, g2 = _suffix_g(h2_ref[...])
    idx = _bin_idx()
    low = jnp.max(jnp.where(g2 >= k2f, idx, -1))
    t_u = (bstar_ref[0] << 16) | low
    t_i = t_u ^ _imin()
    varr = v_ref[...]
    sel = varr > t_i
    eq = varr == t_i
    cnt_gt = jnp.sum(sel.astype(jnp.int32))
    u = varr ^ _imin()
    bits_f = jnp.where(u < 0, u & _imaxp(), jnp.bitwise_not(u))
    dd = jax.lax.bitcast_convert_type(bits_f, jnp.float32)
    ml = _softplus(dd)
    s_main = jnp.sum(jnp.where(sel, ml, 0.0))
    s_eq = jnp.sum(jnp.where(eq, ml, 0.0))
    c_eq = jnp.sum(eq.astype(jnp.float32))
    mlt = s_eq / c_eq
    s_sel = s_main + (k - cnt_gt).astype(jnp.float32) * mlt
    s_sel = jnp.where(k > 0, s_sel, 0.0)
    denom = jnp.maximum((npos_f + k.astype(jnp.float32)), 1.0)
    cls = jnp.clip((s_pos + s_sel) / denom, 0.0, 5.0)
    ver = jnp.clip(ver_sum / jnp.maximum(2.0 * npos_f, 1.0), 0.0, 5.0)
    o_cls[0] = cls
    o_ver[0] = ver
    o_total[0] = cls + ver


def kernel(confidence, predicted_locations, labels, gt_locations):
    b, a = labels.shape
    n = b * a
    nr = n // _LANES
    grid = nr // _BR
    conf = confidence.reshape(n, 2)
    c0 = conf[:, 0].reshape(nr, _LANES)
    c1 = conf[:, 1].reshape(nr, _LANES)
    pl4 = predicted_locations.reshape(n, 4)
    gl4 = gt_locations.reshape(n, 4)
    p1 = pl4[:, 1].reshape(nr, _LANES)
    p3 = pl4[:, 3].reshape(nr, _LANES)
    g1 = gl4[:, 1].reshape(nr, _LANES)
    g3 = gl4[:, 3].reshape(nr, _LANES)
    lab = labels.reshape(nr, _LANES)

    row_spec = pl.BlockSpec((_BR, _LANES), lambda i: (i, 0))
    acc_spec = pl.BlockSpec((8, _LANES), lambda i: (0, 0))
    smem_spec = pl.BlockSpec(memory_space=pltpu.SMEM)
    v, acc, kk = pl.pallas_call(
        functools.partial(_a_body, n, grid),
        grid=(grid,),
        in_specs=[row_spec] * 7,
        out_specs=[row_spec, acc_spec, smem_spec],
        out_shape=[
            jax.ShapeDtypeStruct((nr, _LANES), jnp.int32),
            jax.ShapeDtypeStruct((8, _LANES), jnp.float32),
            jax.ShapeDtypeStruct((1,), jnp.int32),
        ],
    )(c0, c1, lab, p1, p3, g1, g3)

    vflat = v.reshape(n)
    zeros_bins = jnp.zeros((_BINS,), jnp.int32)
    h1 = _sc_hist(_h1_body, n, vflat, zeros_bins)

    vmem_spec = pl.BlockSpec(memory_space=pltpu.VMEM)
    bvec, bstar, chi = pl.pallas_call(
        _f1_body,
        in_specs=[vmem_spec, smem_spec],
        out_specs=[vmem_spec, smem_spec, smem_spec],
        out_shape=[
            jax.ShapeDtypeStruct((8, _LANES), jnp.int32),
            jax.ShapeDtypeStruct((1,), jnp.int32),
            jax.ShapeDtypeStruct((1,), jnp.int32),
        ],
    )(h1.reshape(_NW, 512, _LANES), kk)

    h2 = _sc_hist(_h2_body, n, vflat, zeros_bins, bvec.reshape(8 * _LANES))

    total, cls, ver = pl.pallas_call(
        _bf_body,
        in_specs=[vmem_spec, vmem_spec, vmem_spec, smem_spec, smem_spec, smem_spec],
        out_specs=[smem_spec, smem_spec, smem_spec],
        out_shape=[
            jax.ShapeDtypeStruct((1,), jnp.float32),
            jax.ShapeDtypeStruct((1,), jnp.float32),
            jax.ShapeDtypeStruct((1,), jnp.float32),
        ],
    )(v, acc, h2.reshape(_NW, 512, _LANES), kk, bstar, chi)

    z = jnp.zeros((), jnp.float32)
    return (total.reshape(()), cls.reshape(()), ver.reshape(()), z)


# speedup vs baseline: 1.1371x; 1.1348x over previous
import functools

import jax
import jax.numpy as jnp
from jax import lax
from jax.experimental import pallas as pl
from jax.experimental.pallas import tpu as pltpu
from jax.experimental.pallas import tpu_sc as plsc

_BETA = 1.0 / 9
_NEG_POS_RATIO = 3
_LANES = 128
_BR = 512
_NW = 32
_BINS = 65536


def _imin():
    return jnp.int32(-2147483648)


def _imaxp():
    return jnp.int32(0x7FFFFFFF)


def _softplus(x):
    return jnp.maximum(x, 0.0) + jnp.log1p(jnp.exp(-jnp.abs(x)))


def _a_body(n_total, grid, c0, c1, lab, p1, p3, g1, g3, v_out, acc, k_out):
    step = pl.program_id(0)
    d = c1[...] - c0[...]
    bits = jax.lax.bitcast_convert_type(d, jnp.int32)
    v = jnp.where(bits >= 0, bits, bits ^ _imaxp())
    pos = lab[...] > 0
    v_out[...] = jnp.where(pos, _imin(), v)
    posf = pos.astype(jnp.float32)
    ce_pos = _softplus(-d)
    a1 = jnp.abs(p1[...] - g1[...])
    a3 = jnp.abs(p3[...] - g3[...])
    sl1 = jnp.where(a1 < _BETA, 0.5 * a1 * a1 / _BETA, a1 - 0.5 * _BETA)
    sl3 = jnp.where(a3 < _BETA, 0.5 * a3 * a3 / _BETA, a3 - 0.5 * _BETA)
    pcnt = jnp.sum(posf, axis=0)
    spos = jnp.sum(ce_pos * posf, axis=0)
    verp = jnp.sum((sl1 + sl3) * posf, axis=0)
    rows = jax.lax.broadcasted_iota(jnp.int32, (8, _LANES), 0)
    part = (
        jnp.where(rows == 0, pcnt[None, :], 0.0)
        + jnp.where(rows == 1, spos[None, :], 0.0)
        + jnp.where(rows == 2, verp[None, :], 0.0)
    )

    @pl.when(step == 0)
    def _():
        acc[...] = part

    @pl.when(step != 0)
    def _():
        acc[...] = acc[...] + part

    @pl.when(step == grid - 1)
    def _():
        npos = jnp.sum(acc[0, :]).astype(jnp.int32)
        k_out[0] = jnp.minimum(npos * _NEG_POS_RATIO, jnp.int32(n_total) - npos)


_UNROLL = 8


def _h1_body(n_total, v_hbm, zeros_hbm, out_hbm, data_v, hist_v):
    ch = n_total // _NW
    wid = lax.axis_index("s") * 2 + lax.axis_index("c")
    pltpu.sync_copy(v_hbm.at[pl.ds(wid * ch, ch)], data_v)
    pltpu.sync_copy(zeros_hbm, hist_v)
    ones16 = jnp.full((16,), 1, jnp.int32)
    m31 = jnp.full((16,), -2147483648, jnp.int32)
    sh16 = jnp.full((16,), 16, jnp.int32)

    @plsc.parallel_loop(0, ch // 16, unroll=_UNROLL)
    def hbody(i):
        x = data_v[pl.ds(i * 16, 16)]
        u = lax.bitwise_xor(x, m31)
        hi = lax.shift_right_logical(u, sh16)
        plsc.addupdate_scatter(hist_v, [hi], ones16)

    pltpu.sync_copy(hist_v, out_hbm.at[wid])


def _h2_body(n_total, v_hbm, zeros_hbm, b_hbm, out_hbm, data_v, hist_v, b_v):
    ch = n_total // _NW
    wid = lax.axis_index("s") * 2 + lax.axis_index("c")
    pltpu.sync_copy(v_hbm.at[pl.ds(wid * ch, ch)], data_v)
    pltpu.sync_copy(zeros_hbm, hist_v)
    pltpu.sync_copy(b_hbm.at[pl.ds(0, 16)], b_v)
    bb = b_v[...]
    ones16 = jnp.full((16,), 1, jnp.int32)
    m31 = jnp.full((16,), -2147483648, jnp.int32)
    sh16 = jnp.full((16,), 16, jnp.int32)
    mlow = jnp.full((16,), 0xFFFF, jnp.int32)

    @plsc.parallel_loop(0, ch // 16, unroll=_UNROLL)
    def hbody(i):
        x = data_v[pl.ds(i * 16, 16)]
        u = lax.bitwise_xor(x, m31)
        hi = lax.shift_right_logical(u, sh16)
        low = lax.bitwise_and(u, mlow)
        plsc.addupdate_scatter(hist_v, [low], ones16, mask=hi == bb)

    pltpu.sync_copy(hist_v, out_hbm.at[wid])


def _sc_hist(body, n_total, *args):
    mesh = plsc.VectorSubcoreMesh(core_axis_name="c", subcore_axis_name="s")
    ch = n_total // _NW
    scratch = [
        pltpu.VMEM((ch,), jnp.int32),
        pltpu.VMEM((_BINS,), jnp.int32),
    ]
    if body is _h2_body:
        scratch.append(pltpu.VMEM((16,), jnp.int32))
    return pl.kernel(
        functools.partial(body, n_total),
        mesh=mesh,
        out_type=jax.ShapeDtypeStruct((_NW, _BINS), jnp.int32),
        scratch_types=scratch,
        compiler_params=pltpu.CompilerParams(needs_layout_passes=False),
    )(*args)


def _suffix_g(h_i32):
    m2 = jnp.sum(h_i32.astype(jnp.float32), axis=0)
    ii = jax.lax.broadcasted_iota(jnp.int32, (_LANES, _LANES), 0)
    jj = jax.lax.broadcasted_iota(jnp.int32, (_LANES, _LANES), 1)
    tincl = (ii >= jj).astype(jnp.float32)
    gw = jnp.dot(m2, tincl, preferred_element_type=jnp.float32)
    i5 = jax.lax.broadcasted_iota(jnp.int32, (512, 512), 0)
    j5 = jax.lax.broadcasted_iota(jnp.int32, (512, 512), 1)
    ltr = (j5 > i5).astype(jnp.float32)
    rowtot = gw[:, 0:1]
    sgt = jnp.dot(ltr, rowtot, preferred_element_type=jnp.float32)
    return m2, sgt + gw


def _bin_idx():
    ii = jax.lax.broadcasted_iota(jnp.int32, (512, _LANES), 0)
    jj = jax.lax.broadcasted_iota(jnp.int32, (512, _LANES), 1)
    return ii * _LANES + jj


def _f1_body(h_ref, k_ref, bvec_out, bstar_out, chi_out):
    kf = k_ref[0].astype(jnp.float32)
    m2, g = _suffix_g(h_ref[...])
    idx = _bin_idx()
    bstar = jnp.max(jnp.where(g >= kf, idx, -1))
    chi = jnp.sum(jnp.where(idx > bstar, m2, 0.0)).astype(jnp.int32)
    bvec_out[...] = jnp.full((8, _LANES), bstar, jnp.int32)
    bstar_out[0] = bstar
    chi_out[0] = chi


def _bf_body(v_ref, acc_ref, h2_ref, k_ref, bstar_ref, chi_ref, o_total, o_cls, o_ver):
    acc = acc_ref[...]
    npos_f = jnp.sum(acc[0, :])
    s_pos = jnp.sum(acc[1, :])
    ver_sum = jnp.sum(acc[2, :])
    k = k_ref[0]
    k2f = (k - chi_ref[0]).astype(jnp.float32)
    _, g2 = _suffix_g(h2_ref[...])
    idx = _bin_idx()
    low = jnp.max(jnp.where(g2 >= k2f, idx, -1))
    t_u = (bstar_ref[0] << 16) | low
    t_i = t_u ^ _imin()
    varr = v_ref[...]
    sel = varr > t_i
    eq = varr == t_i
    cnt_gt = jnp.sum(sel.astype(jnp.int32))
    u = varr ^ _imin()
    bits_f = jnp.where(u < 0, u & _imaxp(), jnp.bitwise_not(u))
    dd = jax.lax.bitcast_convert_type(bits_f, jnp.float32)
    ml = _softplus(dd)
    s_main = jnp.sum(jnp.where(sel, ml, 0.0))
    s_eq = jnp.sum(jnp.where(eq, ml, 0.0))
    c_eq = jnp.sum(eq.astype(jnp.float32))
    mlt = s_eq / c_eq
    s_sel = s_main + (k - cnt_gt).astype(jnp.float32) * mlt
    s_sel = jnp.where(k > 0, s_sel, 0.0)
    denom = jnp.maximum((npos_f + k.astype(jnp.float32)), 1.0)
    cls = jnp.clip((s_pos + s_sel) / denom, 0.0, 5.0)
    ver = jnp.clip(ver_sum / jnp.maximum(2.0 * npos_f, 1.0), 0.0, 5.0)
    o_cls[0] = cls
    o_ver[0] = ver
    o_total[0] = cls + ver


def kernel(confidence, predicted_locations, labels, gt_locations):
    b, a = labels.shape
    n = b * a
    nr = n // _LANES
    grid = nr // _BR
    conf = confidence.reshape(n, 2)
    c0 = conf[:, 0].reshape(nr, _LANES)
    c1 = conf[:, 1].reshape(nr, _LANES)
    pl4 = predicted_locations.reshape(n, 4)
    gl4 = gt_locations.reshape(n, 4)
    p1 = pl4[:, 1].reshape(nr, _LANES)
    p3 = pl4[:, 3].reshape(nr, _LANES)
    g1 = gl4[:, 1].reshape(nr, _LANES)
    g3 = gl4[:, 3].reshape(nr, _LANES)
    lab = labels.reshape(nr, _LANES)

    row_spec = pl.BlockSpec((_BR, _LANES), lambda i: (i, 0))
    acc_spec = pl.BlockSpec((8, _LANES), lambda i: (0, 0))
    smem_spec = pl.BlockSpec(memory_space=pltpu.SMEM)
    v, acc, kk = pl.pallas_call(
        functools.partial(_a_body, n, grid),
        grid=(grid,),
        in_specs=[row_spec] * 7,
        out_specs=[row_spec, acc_spec, smem_spec],
        out_shape=[
            jax.ShapeDtypeStruct((nr, _LANES), jnp.int32),
            jax.ShapeDtypeStruct((8, _LANES), jnp.float32),
            jax.ShapeDtypeStruct((1,), jnp.int32),
        ],
    )(c0, c1, lab, p1, p3, g1, g3)

    vflat = v.reshape(n)
    zeros_bins = jnp.zeros((_BINS,), jnp.int32)
    h1 = _sc_hist(_h1_body, n, vflat, zeros_bins)

    vmem_spec = pl.BlockSpec(memory_space=pltpu.VMEM)
    bvec, bstar, chi = pl.pallas_call(
        _f1_body,
        in_specs=[vmem_spec, smem_spec],
        out_specs=[vmem_spec, smem_spec, smem_spec],
        out_shape=[
            jax.ShapeDtypeStruct((8, _LANES), jnp.int32),
            jax.ShapeDtypeStruct((1,), jnp.int32),
            jax.ShapeDtypeStruct((1,), jnp.int32),
        ],
    )(h1.reshape(_NW, 512, _LANES), kk)

    h2 = _sc_hist(_h2_body, n, vflat, zeros_bins, bvec.reshape(8 * _LANES))

    total, cls, ver = pl.pallas_call(
        _bf_body,
        in_specs=[vmem_spec, vmem_spec, vmem_spec, smem_spec, smem_spec, smem_spec],
        out_specs=[smem_spec, smem_spec, smem_spec],
        out_shape=[
            jax.ShapeDtypeStruct((1,), jnp.float32),
            jax.ShapeDtypeStruct((1,), jnp.float32),
            jax.ShapeDtypeStruct((1,), jnp.float32),
        ],
    )(v, acc, h2.reshape(_NW, 512, _LANES), kk, bstar, chi)

    z = jnp.zeros((), jnp.float32)
    return (total.reshape(()), cls.reshape(()), ver.reshape(()), z)


# speedup vs baseline: 2.2885x; 2.0125x over previous
import functools

import jax
import jax.numpy as jnp
from jax.experimental import pallas as pl
from jax.experimental.pallas import tpu as pltpu

_BETA = 1.0 / 9
_NEG_POS_RATIO = 3
_LANES = 128
_BR = 512
def _imin():
    return jnp.int32(-2147483648)


def _imaxp():
    return jnp.int32(0x7FFFFFFF)


def _softplus(x):
    return jnp.maximum(x, 0.0) + jnp.log1p(jnp.exp(-jnp.abs(x)))


def _a_body(c0, c1, lab, p1, p3, g1, g3, v_out, acc):
    step = pl.program_id(0)
    d = c1[...] - c0[...]
    bits = jax.lax.bitcast_convert_type(d, jnp.int32)
    v = jnp.where(bits >= 0, bits, bits ^ _imaxp())
    pos = lab[...] > 0
    v_out[...] = jnp.where(pos, _imin(), v)
    posf = pos.astype(jnp.float32)
    ce_pos = _softplus(-d)
    a1 = jnp.abs(p1[...] - g1[...])
    a3 = jnp.abs(p3[...] - g3[...])
    sl1 = jnp.where(a1 < _BETA, 0.5 * a1 * a1 / _BETA, a1 - 0.5 * _BETA)
    sl3 = jnp.where(a3 < _BETA, 0.5 * a3 * a3 / _BETA, a3 - 0.5 * _BETA)
    pcnt = jnp.sum(posf, axis=0)
    spos = jnp.sum(ce_pos * posf, axis=0)
    verp = jnp.sum((sl1 + sl3) * posf, axis=0)
    rows = jax.lax.broadcasted_iota(jnp.int32, (8, _LANES), 0)
    part = (
        jnp.where(rows == 0, pcnt[None, :], 0.0)
        + jnp.where(rows == 1, spos[None, :], 0.0)
        + jnp.where(rows == 2, verp[None, :], 0.0)
    )

    @pl.when(step == 0)
    def _():
        acc[...] = part

    @pl.when(step != 0)
    def _():
        acc[...] = acc[...] + part


def _b_body(n_total, v_ref, acc_ref, o_total, o_cls, o_ver):
    acc = acc_ref[...]
    npos_f = jnp.sum(acc[0, :])
    s_pos = jnp.sum(acc[1, :])
    ver_sum = jnp.sum(acc[2, :])
    npos = npos_f.astype(jnp.int32)
    k = jnp.minimum(npos * _NEG_POS_RATIO, jnp.int32(n_total) - npos)
    varr = v_ref[...]
    v3 = varr.reshape(16, 512, _LANES)

    def bs_body(i, t_u):
        cand = t_u | (jnp.int32(1) << (31 - i))
        cand_i = cand ^ _imin()
        part = jnp.sum((v3 >= cand_i).astype(jnp.int32), axis=1)
        cnt = jnp.sum(part)
        return jnp.where(cnt >= k, cand, t_u)

    t_u = jax.lax.fori_loop(0, 32, bs_body, jnp.int32(0), unroll=False)
    t_i = t_u ^ _imin()
    sel = varr > t_i
    eq = varr == t_i
    cnt_gt = jnp.sum(sel.astype(jnp.int32))
    u = varr ^ _imin()
    bits_f = jnp.where(u < 0, u & _imaxp(), jnp.bitwise_not(u))
    dd = jax.lax.bitcast_convert_type(bits_f, jnp.float32)
    ml = _softplus(dd)
    s_main = jnp.sum(jnp.where(sel, ml, 0.0))
    s_eq = jnp.sum(jnp.where(eq, ml, 0.0))
    c_eq = jnp.sum(eq.astype(jnp.float32))
    mlt = s_eq / c_eq
    s_sel = s_main + (k - cnt_gt).astype(jnp.float32) * mlt
    s_sel = jnp.where(k > 0, s_sel, 0.0)
    denom = jnp.maximum((npos + k).astype(jnp.float32), 1.0)
    cls = jnp.clip((s_pos + s_sel) / denom, 0.0, 5.0)
    ver = jnp.clip(ver_sum / jnp.maximum(2.0 * npos_f, 1.0), 0.0, 5.0)
    o_cls[0] = cls
    o_ver[0] = ver
    o_total[0] = cls + ver


def kernel(confidence, predicted_locations, labels, gt_locations):
    b, a = labels.shape
    n = b * a
    nr = n // _LANES
    grid = nr // _BR
    conf = confidence.reshape(n, 2)
    c0 = conf[:, 0].reshape(nr, _LANES)
    c1 = conf[:, 1].reshape(nr, _LANES)
    pl4 = predicted_locations.reshape(n, 4)
    gl4 = gt_locations.reshape(n, 4)
    p1 = pl4[:, 1].reshape(nr, _LANES)
    p3 = pl4[:, 3].reshape(nr, _LANES)
    g1 = gl4[:, 1].reshape(nr, _LANES)
    g3 = gl4[:, 3].reshape(nr, _LANES)
    lab = labels.reshape(nr, _LANES)

    row_spec = pl.BlockSpec((_BR, _LANES), lambda i: (i, 0))
    acc_spec = pl.BlockSpec((8, _LANES), lambda i: (0, 0))
    v, acc = pl.pallas_call(
        _a_body,
        grid=(grid,),
        in_specs=[row_spec] * 7,
        out_specs=[row_spec, acc_spec],
        out_shape=[
            jax.ShapeDtypeStruct((nr, _LANES), jnp.int32),
            jax.ShapeDtypeStruct((8, _LANES), jnp.float32),
        ],
    )(c0, c1, lab, p1, p3, g1, g3)

    total, cls, ver = pl.pallas_call(
        functools.partial(_b_body, n),
        in_specs=[
            pl.BlockSpec(memory_space=pltpu.VMEM),
            pl.BlockSpec(memory_space=pltpu.VMEM),
        ],
        out_specs=[
            pl.BlockSpec(memory_space=pltpu.SMEM),
            pl.BlockSpec(memory_space=pltpu.SMEM),
            pl.BlockSpec(memory_space=pltpu.SMEM),
        ],
        out_shape=[
            jax.ShapeDtypeStruct((1,), jnp.float32),
            jax.ShapeDtypeStruct((1,), jnp.float32),
            jax.ShapeDtypeStruct((1,), jnp.float32),
        ],
    )(v, acc)

    z = jnp.zeros((), jnp.float32)
    return (total.reshape(()), cls.reshape(()), ver.reshape(()), z)
